# baseline (speedup 1.0000x reference)
import jax
import jax.numpy as jnp
from jax import lax
from jax.experimental import pallas as pl
from jax.experimental.pallas import tpu as pltpu
from jax.experimental.pallas import tpu_sc as plsc

N, E, D, H, S, C, G = 10000, 320000, 128, 128, 64, 16, 64

_ROWS = 2000

_SC_NC, _SC_NS = 2, 16
_NW = _SC_NC * _SC_NS
_K = 128
_CHUNKS = E // _K
_VW = 2 * _NW
_SPC = 39
_XTRA = _CHUNKS - _VW * _SPC
_NP = 10112
_STRIPE = _NP // _SC_NS


def _sc_agg_body(h_hbm, src3_hbm, dst3_hbm, srcf_hbm, dstf_hbm,
                 agg_out, idx_v, dst2_v, idxx_v, dstx_v,
                 buf_a, buf_b, sh_agg, sem_a, sem_b):
    c = lax.axis_index("c")
    s = lax.axis_index("s")
    wid = s * _SC_NC + c

    zv = jnp.zeros((16,), jnp.float32)

    def zrow(i, carry):
        buf_a[lax.div(i, 8), pl.ds(lax.rem(i, 8) * 16, 16)] = zv
        return carry
    lax.fori_loop(0, _K * 8, zrow, 0)

    def zstripe(j, carry):
        pltpu.sync_copy(buf_a.at[pl.ds(0, _STRIPE // 8)],
                        sh_agg.at[pl.ds(s * _STRIPE + j * (_STRIPE // 8),
                                        _STRIPE // 8)])
        return carry
    lax.fori_loop(0, 8, zstripe, 0)

    plsc.subcore_barrier()

    def gather(j, buf, sem):
        pltpu.async_copy(h_hbm.at[idx_v.at[j]], buf, sem)

    def gwait(j, buf, sem):
        pltpu.make_async_copy(h_hbm.at[idx_v.at[j]], buf, sem).wait()

    for b in range(2):
        pltpu.sync_copy(src3_hbm.at[2 * wid + b], idx_v)
        pltpu.sync_copy(dst3_hbm.at[2 * wid + b], dst2_v)
        gather(0, buf_a, sem_a)
        gather(1, buf_b, sem_b)

        def pair(k, carry):
            j0 = 2 * k
            gwait(j0, buf_a, sem_a)
            pltpu.sync_copy(buf_a, sh_agg.at[dst2_v.at[j0]], add=True)
            gather(j0 + 2, buf_a, sem_a)

            gwait(j0 + 1, buf_b, sem_b)
            pltpu.sync_copy(buf_b, sh_agg.at[dst2_v.at[j0 + 1]], add=True)

            @pl.when(k < _SPC // 2 - 1)
            def _gb():
                gather(j0 + 3, buf_b, sem_b)

            return carry

        lax.fori_loop(0, _SPC // 2, pair, 0)
        gwait(_SPC - 1, buf_a, sem_a)
        pltpu.sync_copy(buf_a, sh_agg.at[dst2_v.at[_SPC - 1]], add=True)

    @pl.when(wid < _XTRA)
    def _extra():
        base = (_VW * _SPC + wid) * _K
        pltpu.sync_copy(srcf_hbm.at[pl.ds(base, _K)], idxx_v)
        pltpu.sync_copy(dstf_hbm.at[pl.ds(base, _K)], dstx_v)
        pltpu.async_copy(h_hbm.at[idxx_v], buf_a, sem_a).wait()
        pltpu.sync_copy(buf_a, sh_agg.at[dstx_v], add=True)

    plsc.subcore_barrier()
    pltpu.sync_copy(sh_agg.at[pl.ds(s * _STRIPE, _STRIPE)],
                    agg_out.at[c, pl.ds(s * _STRIPE, _STRIPE)])


@jax.jit
def _sc_agg(h, src3, dst3, srcf, dstf):
    mesh = plsc.VectorSubcoreMesh(core_axis_name="c", subcore_axis_name="s")
    return pl.kernel(
        _sc_agg_body,
        out_type=jax.ShapeDtypeStruct((_SC_NC, _NP, D), jnp.float32),
        mesh=mesh,
        scratch_types=(
            pltpu.VMEM((_SPC, _K), jnp.int32),
            pltpu.VMEM((_SPC, _K), jnp.int32),
            pltpu.VMEM((_K,), jnp.int32),
            pltpu.VMEM((_K,), jnp.int32),
            pltpu.VMEM((_K, D), jnp.float32),
            pltpu.VMEM((_K, D), jnp.float32),
            pltpu.VMEM_SHARED((_NP, D), jnp.float32),
            pltpu.SemaphoreType.DMA,
            pltpu.SemaphoreType.DMA,
        ),
    )(h, src3, dst3, srcf, dstf)


_EB = 12800
_DR = _NP // _K


def _bndeg_body(dst_ref, x_ref, g_ref, b_ref, h_ref, deg_ref, acc_ref):
    i = pl.program_id(0)

    @pl.when(i == 0)
    def _init():
        acc_ref[...] = jnp.zeros((_DR, _K), jnp.float32)

    d = dst_ref[...]
    q = lax.shift_right_logical(d, 7)
    r = lax.bitwise_and(d, 127)
    oq = (q == lax.broadcasted_iota(jnp.int32, (1, _DR), 1)
          ).astype(jnp.bfloat16)
    orr = (r == lax.broadcasted_iota(jnp.int32, (1, _K), 1)
           ).astype(jnp.bfloat16)
    acc_ref[...] += lax.dot_general(oq, orr, (((0,), (0,)), ((), ())),
                                    preferred_element_type=jnp.float32)

    @pl.when(i == pl.num_programs(0) - 1)
    def _fin():
        deg_ref[...] = acc_ref[...]
        x = x_ref[...]
        mu = jnp.mean(x, axis=0, keepdims=True)
        var = jnp.mean((x - mu) ** 2, axis=0, keepdims=True)
        h_ref[...] = ((x - mu) / jnp.sqrt(var + 1e-5) * g_ref[...]
                      + b_ref[...])


def _bndeg(dstc, x, gamma, beta):
    return pl.pallas_call(
        _bndeg_body,
        grid=(E // _EB,),
        in_specs=[
            pl.BlockSpec((_EB, 1), lambda i: (i, 0)),
            pl.BlockSpec((N, D), lambda i: (0, 0)),
            pl.BlockSpec((1, D), lambda i: (0, 0)),
            pl.BlockSpec((1, D), lambda i: (0, 0)),
        ],
        out_specs=[
            pl.BlockSpec((N, D), lambda i: (0, 0)),
            pl.BlockSpec((_DR, _K), lambda i: (0, 0)),
        ],
        out_shape=[
            jax.ShapeDtypeStruct((N, D), jnp.float32),
            jax.ShapeDtypeStruct((_DR, _K), jnp.float32),
        ],
        scratch_shapes=[pltpu.VMEM((_DR, _K), jnp.float32)],
    )(dstc, x, gamma.reshape(1, D), beta.reshape(1, D))


def _combine_body(p_ref, deg_ref, h_ref, wl_ref, bl_ref, wr_ref, o_ref):
    agg = jnp.sum(p_ref[...], axis=0)
    zl = lax.dot_general(agg, wl_ref[...], (((1,), (1,)), ((), ())),
                         preferred_element_type=jnp.float32)
    zr = lax.dot_general(h_ref[...], wr_ref[...], (((1,), (1,)), ((), ())),
                         preferred_element_type=jnp.float32)
    deg = jnp.maximum(deg_ref[...], 1.0)
    z = zl / deg + bl_ref[...] + zr
    o_ref[...] = jnp.where(z > 0, z, jnp.exp(jnp.minimum(z, 0.0)) - 1.0)


def _combine(partials, deg, h, Wl, bl, Wr):
    P = partials.shape[0]
    grid = N // _ROWS
    return pl.pallas_call(
        _combine_body,
        grid=(grid,),
        in_specs=[
            pl.BlockSpec((P, _ROWS, D), lambda i: (0, i, 0)),
            pl.BlockSpec((_ROWS, 1), lambda i: (i, 0)),
            pl.BlockSpec((_ROWS, D), lambda i: (i, 0)),
            pl.BlockSpec((H, D), lambda i: (0, 0)),
            pl.BlockSpec((1, H), lambda i: (0, 0)),
            pl.BlockSpec((H, D), lambda i: (0, 0)),
        ],
        out_specs=pl.BlockSpec((_ROWS, H), lambda i: (i, 0)),
        out_shape=jax.ShapeDtypeStruct((N, H), jnp.float32),
    )(partials, deg, h, Wl, bl.reshape(1, H), Wr)


def _final_body(p_ref, deg_ref, h_ref, wl_ref, bl_ref, wr_ref,
                batch_ref, wc1_ref, bc1_ref, wc2_ref, bc2_ref,
                out_ref, rv_ref, acc_ref):
    i = pl.program_id(0)
    nsteps = pl.num_programs(0)

    @pl.when(i == 0)
    def _init():
        acc_ref[...] = jnp.full((G, H), -jnp.inf, jnp.float32)

    agg = jnp.sum(p_ref[...], axis=0)
    zl = lax.dot_general(agg, wl_ref[...], (((1,), (1,)), ((), ())),
                         preferred_element_type=jnp.float32)
    zr = lax.dot_general(h_ref[...], wr_ref[...], (((1,), (1,)), ((), ())),
                         preferred_element_type=jnp.float32)
    deg = jnp.maximum(deg_ref[...], 1.0)
    z = zl / deg + bl_ref[...] + zr
    h2 = jnp.where(z > 0, z, jnp.exp(jnp.minimum(z, 0.0)) - 1.0)

    batch = batch_ref[...]
    gmin = jnp.min(batch)
    gmax = jnp.max(batch)

    def body(g, _):
        m = batch == g
        val = jnp.max(jnp.where(m, h2, -jnp.inf), axis=0, keepdims=True)
        cur = acc_ref[pl.ds(g, 1), :]
        acc_ref[pl.ds(g, 1), :] = jnp.maximum(cur, val)
        return 0

    lax.fori_loop(gmin, gmax + 1, body, 0)

    @pl.when(i == nsteps - 1)
    def _fin():
        acc = acc_ref[...]
        read_out = jnp.where(jnp.isfinite(acc), acc, 0.0)
        rv = lax.dot_general(read_out, wc1_ref[...], (((1,), (1,)), ((), ())),
                             preferred_element_type=jnp.float32) + bc1_ref[...]
        out = lax.dot_general(rv, wc2_ref[...], (((1,), (1,)), ((), ())),
                              preferred_element_type=jnp.float32) + bc2_ref[...]
        rv_ref[...] = rv
        out_ref[...] = out


def _final(partials, deg, h, Wl, bl, Wr, batch, Wc1, bc1, Wc2, bc2):
    P = partials.shape[0]
    grid = N // _ROWS
    return pl.pallas_call(
        _final_body,
        grid=(grid,),
        in_specs=[
            pl.BlockSpec((P, _ROWS, D), lambda i: (0, i, 0)),
            pl.BlockSpec((_ROWS, 1), lambda i: (i, 0)),
            pl.BlockSpec((_ROWS, D), lambda i: (i, 0)),
            pl.BlockSpec((H, H), lambda i: (0, 0)),
            pl.BlockSpec((1, H), lambda i: (0, 0)),
            pl.BlockSpec((H, H), lambda i: (0, 0)),
            pl.BlockSpec((_ROWS, 1), lambda i: (i, 0)),
            pl.BlockSpec((S, H), lambda i: (0, 0)),
            pl.BlockSpec((1, S), lambda i: (0, 0)),
            pl.BlockSpec((C, S), lambda i: (0, 0)),
            pl.BlockSpec((1, C), lambda i: (0, 0)),
        ],
        out_specs=[
            pl.BlockSpec((G, C), lambda i: (0, 0)),
            pl.BlockSpec((G, S), lambda i: (0, 0)),
        ],
        out_shape=[
            jax.ShapeDtypeStruct((G, C), jnp.float32),
            jax.ShapeDtypeStruct((G, S), jnp.float32),
        ],
        scratch_shapes=[pltpu.VMEM((G, H), jnp.float32)],
    )(partials, deg, h, Wl, bl.reshape(1, H), Wr, batch,
      Wc1, bc1.reshape(1, S), Wc2, bc2.reshape(1, C))


def kernel(x, edge_index, batch, bn_gamma, bn_beta,
           W1l, b1l, W1r, W2l, b2l, W2r, Wc1, bc1, Wc2, bc2):
    srcf = edge_index[0]
    dstf = edge_index[1]
    main = _VW * _SPC * _K
    src3 = srcf[:main].reshape(_VW, _SPC, _K)
    dst3 = dstf[:main].reshape(_VW, _SPC, _K)

    h0, degm = _bndeg(dstf[:, None], x, bn_gamma, bn_beta)
    degf = degm.reshape(_NP)[:N, None]

    agg1 = _sc_agg(h0, src3, dst3, srcf, dstf)
    h1 = _combine(agg1, degf, h0, W1l, b1l, W1r)
    agg2 = _sc_agg(h1, src3, dst3, srcf, dstf)
    out, rv = _final(agg2, degf, h1, W2l, b2l, W2r, batch[:, None],
                     Wc1, bc1, Wc2, bc2)
    return (out, rv)


# speedup vs baseline: 1.0885x; 1.0885x over previous
import jax
import jax.numpy as jnp
from jax import lax
from jax.experimental import pallas as pl
from jax.experimental.pallas import tpu as pltpu
from jax.experimental.pallas import tpu_sc as plsc

N, E, D, H, S, C, G = 10000, 320000, 128, 128, 64, 16, 64

_ROWS = 2000

_SC_NC, _SC_NS = 2, 16
_NW = _SC_NC * _SC_NS
_K = 128
_CHUNKS = E // _K
_VW = 3 * _NW
_SPC = 26
_XTRA = _CHUNKS - _VW * _SPC
_NP = 10112
_STRIPE = _NP // _SC_NS


def _sc_agg_body(h_hbm, src3_hbm, dst3_hbm, srcf_hbm, dstf_hbm,
                 agg_out, idx_v, dst2_v, idxx_v, dstx_v,
                 buf_a, buf_b, sh_agg, sem_a, sem_b, sem_sa, sem_sb):
    c = lax.axis_index("c")
    s = lax.axis_index("s")
    wid = s * _SC_NC + c

    zv = jnp.zeros((16,), jnp.float32)

    def zrow(i, carry):
        buf_a[lax.div(i, 8), pl.ds(lax.rem(i, 8) * 16, 16)] = zv
        return carry
    lax.fori_loop(0, _K * 8, zrow, 0)

    def zstripe(j, carry):
        pltpu.sync_copy(buf_a.at[pl.ds(0, _STRIPE // 8)],
                        sh_agg.at[pl.ds(s * _STRIPE + j * (_STRIPE // 8),
                                        _STRIPE // 8)])
        return carry
    lax.fori_loop(0, 8, zstripe, 0)

    plsc.subcore_barrier()

    def gather(j, buf, sem):
        pltpu.async_copy(h_hbm.at[idx_v.at[j]], buf, sem)

    def gwait(j, buf, sem):
        pltpu.make_async_copy(h_hbm.at[idx_v.at[j]], buf, sem).wait()

    def scat(j, buf, sem):
        pltpu.async_copy(buf, sh_agg.at[dst2_v.at[j]], sem, add=True)

    def swait(j, buf, sem):
        pltpu.make_async_copy(buf, sh_agg.at[dst2_v.at[j]], sem).wait()

    for b in range(3):
        pltpu.sync_copy(src3_hbm.at[3 * wid + b], idx_v)
        pltpu.sync_copy(dst3_hbm.at[3 * wid + b], dst2_v)
        gather(0, buf_a, sem_a)
        gather(1, buf_b, sem_b)

        def pair(k, carry):
            j0 = 2 * k
            gwait(j0, buf_a, sem_a)
            scat(j0, buf_a, sem_sa)
            gwait(j0 + 1, buf_b, sem_b)
            scat(j0 + 1, buf_b, sem_sb)
            swait(j0, buf_a, sem_sa)

            @pl.when(k < _SPC // 2 - 1)
            def _ga():
                gather(j0 + 2, buf_a, sem_a)

            swait(j0 + 1, buf_b, sem_sb)

            @pl.when(k < _SPC // 2 - 1)
            def _gb():
                gather(j0 + 3, buf_b, sem_b)

            return carry

        lax.fori_loop(0, _SPC // 2, pair, 0)

    @pl.when(wid < _XTRA)
    def _extra():
        base = (_VW * _SPC + wid) * _K
        pltpu.sync_copy(srcf_hbm.at[pl.ds(base, _K)], idxx_v)
        pltpu.sync_copy(dstf_hbm.at[pl.ds(base, _K)], dstx_v)
        pltpu.async_copy(h_hbm.at[idxx_v], buf_a, sem_a).wait()
        pltpu.sync_copy(buf_a, sh_agg.at[dstx_v], add=True)

    plsc.subcore_barrier()
    pltpu.sync_copy(sh_agg.at[pl.ds(s * _STRIPE, _STRIPE)],
                    agg_out.at[c, pl.ds(s * _STRIPE, _STRIPE)])


@jax.jit
def _sc_agg(h, src3, dst3, srcf, dstf):
    mesh = plsc.VectorSubcoreMesh(core_axis_name="c", subcore_axis_name="s")
    return pl.kernel(
        _sc_agg_body,
        out_type=jax.ShapeDtypeStruct((_SC_NC, _NP, D), jnp.float32),
        mesh=mesh,
        scratch_types=(
            pltpu.VMEM((_SPC, _K), jnp.int32),
            pltpu.VMEM((_SPC, _K), jnp.int32),
            pltpu.VMEM((_K,), jnp.int32),
            pltpu.VMEM((_K,), jnp.int32),
            pltpu.VMEM((_K, D), jnp.float32),
            pltpu.VMEM((_K, D), jnp.float32),
            pltpu.VMEM_SHARED((_NP, D), jnp.float32),
            pltpu.SemaphoreType.DMA,
            pltpu.SemaphoreType.DMA,
            pltpu.SemaphoreType.DMA,
            pltpu.SemaphoreType.DMA,
        ),
    )(h, src3, dst3, srcf, dstf)


_EB = 12800
_DR = _NP // _K


def _deg_body(dst_ref, o_ref, acc_ref):
    i = pl.program_id(0)

    @pl.when(i == 0)
    def _init():
        acc_ref[...] = jnp.zeros((_DR, _K), jnp.float32)

    d = dst_ref[...]
    q = lax.shift_right_logical(d, 7)
    r = lax.bitwise_and(d, 127)
    oq = (q == lax.broadcasted_iota(jnp.int32, (1, _DR), 1)
          ).astype(jnp.bfloat16)
    orr = (r == lax.broadcasted_iota(jnp.int32, (1, _K), 1)
           ).astype(jnp.bfloat16)
    acc_ref[...] += lax.dot_general(oq, orr, (((0,), (0,)), ((), ())),
                                    preferred_element_type=jnp.float32)

    @pl.when(i == pl.num_programs(0) - 1)
    def _fin():
        o_ref[...] = acc_ref[...]


def _deg(dstc):
    return pl.pallas_call(
        _deg_body,
        grid=(E // _EB,),
        in_specs=[pl.BlockSpec((_EB, 1), lambda i: (i, 0))],
        out_specs=pl.BlockSpec((_DR, _K), lambda i: (0, 0)),
        out_shape=jax.ShapeDtypeStruct((_DR, _K), jnp.float32),
        scratch_shapes=[pltpu.VMEM((_DR, _K), jnp.float32)],
    )(dstc)


def _bn_body(x_ref, g_ref, b_ref, o_ref):
    x = x_ref[...]
    mu = jnp.mean(x, axis=0, keepdims=True)
    var = jnp.mean((x - mu) ** 2, axis=0, keepdims=True)
    o_ref[...] = (x - mu) / jnp.sqrt(var + 1e-5) * g_ref[...] + b_ref[...]


def _bn(x, gamma, beta):
    return pl.pallas_call(
        _bn_body,
        out_shape=jax.ShapeDtypeStruct((N, D), jnp.float32),
    )(x, gamma.reshape(1, D), beta.reshape(1, D))


def _combine_body(p_ref, deg_ref, h_ref, wl_ref, bl_ref, wr_ref, o_ref):
    agg = jnp.sum(p_ref[...], axis=0)
    zl = lax.dot_general(agg, wl_ref[...], (((1,), (1,)), ((), ())),
                         preferred_element_type=jnp.float32)
    zr = lax.dot_general(h_ref[...], wr_ref[...], (((1,), (1,)), ((), ())),
                         preferred_element_type=jnp.float32)
    deg = jnp.maximum(deg_ref[...], 1.0)
    z = zl / deg + bl_ref[...] + zr
    o_ref[...] = jnp.where(z > 0, z, jnp.exp(jnp.minimum(z, 0.0)) - 1.0)


def _combine(partials, deg, h, Wl, bl, Wr):
    P = partials.shape[0]
    grid = N // _ROWS
    return pl.pallas_call(
        _combine_body,
        grid=(grid,),
        in_specs=[
            pl.BlockSpec((P, _ROWS, D), lambda i: (0, i, 0)),
            pl.BlockSpec((_ROWS, 1), lambda i: (i, 0)),
            pl.BlockSpec((_ROWS, D), lambda i: (i, 0)),
            pl.BlockSpec((H, D), lambda i: (0, 0)),
            pl.BlockSpec((1, H), lambda i: (0, 0)),
            pl.BlockSpec((H, D), lambda i: (0, 0)),
        ],
        out_specs=pl.BlockSpec((_ROWS, H), lambda i: (i, 0)),
        out_shape=jax.ShapeDtypeStruct((N, H), jnp.float32),
    )(partials, deg, h, Wl, bl.reshape(1, H), Wr)


def _final_body(p_ref, deg_ref, h_ref, wl_ref, bl_ref, wr_ref,
                batch_ref, wc1_ref, bc1_ref, wc2_ref, bc2_ref,
                out_ref, rv_ref, acc_ref):
    i = pl.program_id(0)
    nsteps = pl.num_programs(0)

    @pl.when(i == 0)
    def _init():
        acc_ref[...] = jnp.full((G, H), -jnp.inf, jnp.float32)

    agg = jnp.sum(p_ref[...], axis=0)
    zl = lax.dot_general(agg, wl_ref[...], (((1,), (1,)), ((), ())),
                         preferred_element_type=jnp.float32)
    zr = lax.dot_general(h_ref[...], wr_ref[...], (((1,), (1,)), ((), ())),
                         preferred_element_type=jnp.float32)
    deg = jnp.maximum(deg_ref[...], 1.0)
    z = zl / deg + bl_ref[...] + zr
    h2 = jnp.where(z > 0, z, jnp.exp(jnp.minimum(z, 0.0)) - 1.0)

    batch = batch_ref[...]
    gmin = jnp.min(batch)
    gmax = jnp.max(batch)

    def body(g, _):
        m = batch == g
        val = jnp.max(jnp.where(m, h2, -jnp.inf), axis=0, keepdims=True)
        cur = acc_ref[pl.ds(g, 1), :]
        acc_ref[pl.ds(g, 1), :] = jnp.maximum(cur, val)
        return 0

    lax.fori_loop(gmin, gmax + 1, body, 0)

    @pl.when(i == nsteps - 1)
    def _fin():
        acc = acc_ref[...]
        read_out = jnp.where(jnp.isfinite(acc), acc, 0.0)
        rv = lax.dot_general(read_out, wc1_ref[...], (((1,), (1,)), ((), ())),
                             preferred_element_type=jnp.float32) + bc1_ref[...]
        out = lax.dot_general(rv, wc2_ref[...], (((1,), (1,)), ((), ())),
                              preferred_element_type=jnp.float32) + bc2_ref[...]
        rv_ref[...] = rv
        out_ref[...] = out


def _final(partials, deg, h, Wl, bl, Wr, batch, Wc1, bc1, Wc2, bc2):
    P = partials.shape[0]
    grid = N // _ROWS
    return pl.pallas_call(
        _final_body,
        grid=(grid,),
        in_specs=[
            pl.BlockSpec((P, _ROWS, D), lambda i: (0, i, 0)),
            pl.BlockSpec((_ROWS, 1), lambda i: (i, 0)),
            pl.BlockSpec((_ROWS, D), lambda i: (i, 0)),
            pl.BlockSpec((H, H), lambda i: (0, 0)),
            pl.BlockSpec((1, H), lambda i: (0, 0)),
            pl.BlockSpec((H, H), lambda i: (0, 0)),
            pl.BlockSpec((_ROWS, 1), lambda i: (i, 0)),
            pl.BlockSpec((S, H), lambda i: (0, 0)),
            pl.BlockSpec((1, S), lambda i: (0, 0)),
            pl.BlockSpec((C, S), lambda i: (0, 0)),
            pl.BlockSpec((1, C), lambda i: (0, 0)),
        ],
        out_specs=[
            pl.BlockSpec((G, C), lambda i: (0, 0)),
            pl.BlockSpec((G, S), lambda i: (0, 0)),
        ],
        out_shape=[
            jax.ShapeDtypeStruct((G, C), jnp.float32),
            jax.ShapeDtypeStruct((G, S), jnp.float32),
        ],
        scratch_shapes=[pltpu.VMEM((G, H), jnp.float32)],
    )(partials, deg, h, Wl, bl.reshape(1, H), Wr, batch,
      Wc1, bc1.reshape(1, S), Wc2, bc2.reshape(1, C))


def kernel(x, edge_index, batch, bn_gamma, bn_beta,
           W1l, b1l, W1r, W2l, b2l, W2r, Wc1, bc1, Wc2, bc2):
    srcf = edge_index[0]
    dstf = edge_index[1]
    main = _VW * _SPC * _K
    src3 = srcf[:main].reshape(_VW, _SPC, _K)
    dst3 = dstf[:main].reshape(_VW, _SPC, _K)

    degf = _deg(dstf[:, None]).reshape(_NP)[:N, None]

    h0 = _bn(x, bn_gamma, bn_beta)
    agg1 = _sc_agg(h0, src3, dst3, srcf, dstf)
    h1 = _combine(agg1, degf, h0, W1l, b1l, W1r)
    agg2 = _sc_agg(h1, src3, dst3, srcf, dstf)
    out, rv = _final(agg2, degf, h1, W2l, b2l, W2r, batch[:, None],
                     Wc1, bc1, Wc2, bc2)
    return (out, rv)


# speedup vs baseline: 1.1731x; 1.0777x over previous
import jax
import jax.numpy as jnp
from jax import lax
from jax.experimental import pallas as pl
from jax.experimental.pallas import tpu as pltpu
from jax.experimental.pallas import tpu_sc as plsc

N, E, D, H, S, C, G = 10000, 320000, 128, 128, 64, 16, 64

_ROWS = 2000

_SC_NC, _SC_NS = 2, 16
_NW = _SC_NC * _SC_NS
_K = 128
_CHUNKS = E // _K
_VW = 2 * _NW
_SPC = 39
_XTRA = _CHUNKS - _VW * _SPC
_NP = 10112
_STRIPE = _NP // _SC_NS


def _sc_agg_body(h_hbm, src3_hbm, dst3_hbm, srcf_hbm, dstf_hbm,
                 agg_out, idx_v, dst2_v, idxx_v, dstx_v,
                 buf_a, buf_b, sh_agg, sem_a, sem_b):
    c = lax.axis_index("c")
    s = lax.axis_index("s")
    wid = s * _SC_NC + c

    zv = jnp.zeros((16,), jnp.float32)

    def zrow(i, carry):
        buf_a[lax.div(i, 8), pl.ds(lax.rem(i, 8) * 16, 16)] = zv
        return carry
    lax.fori_loop(0, _K * 8, zrow, 0)

    def zstripe(j, carry):
        pltpu.sync_copy(buf_a.at[pl.ds(0, _STRIPE // 8)],
                        sh_agg.at[pl.ds(s * _STRIPE + j * (_STRIPE // 8),
                                        _STRIPE // 8)])
        return carry
    lax.fori_loop(0, 8, zstripe, 0)

    plsc.subcore_barrier()

    def gather(j, buf, sem):
        pltpu.async_copy(h_hbm.at[idx_v.at[j]], buf, sem)

    def gwait(j, buf, sem):
        pltpu.make_async_copy(h_hbm.at[idx_v.at[j]], buf, sem).wait()

    for b in range(2):
        pltpu.sync_copy(src3_hbm.at[2 * wid + b], idx_v)
        pltpu.sync_copy(dst3_hbm.at[2 * wid + b], dst2_v)
        gather(0, buf_a, sem_a)
        gather(1, buf_b, sem_b)

        def pair(k, carry):
            j0 = 2 * k
            gwait(j0, buf_a, sem_a)
            pltpu.sync_copy(buf_a, sh_agg.at[dst2_v.at[j0]], add=True)
            gather(j0 + 2, buf_a, sem_a)

            gwait(j0 + 1, buf_b, sem_b)
            pltpu.sync_copy(buf_b, sh_agg.at[dst2_v.at[j0 + 1]], add=True)

            @pl.when(k < _SPC // 2 - 1)
            def _gb():
                gather(j0 + 3, buf_b, sem_b)

            return carry

        lax.fori_loop(0, _SPC // 2, pair, 0)
        gwait(_SPC - 1, buf_a, sem_a)
        pltpu.sync_copy(buf_a, sh_agg.at[dst2_v.at[_SPC - 1]], add=True)

    @pl.when(wid < _XTRA)
    def _extra():
        base = (_VW * _SPC + wid) * _K
        pltpu.sync_copy(srcf_hbm.at[pl.ds(base, _K)], idxx_v)
        pltpu.sync_copy(dstf_hbm.at[pl.ds(base, _K)], dstx_v)
        pltpu.async_copy(h_hbm.at[idxx_v], buf_a, sem_a).wait()
        pltpu.sync_copy(buf_a, sh_agg.at[dstx_v], add=True)

    plsc.subcore_barrier()
    pltpu.sync_copy(sh_agg.at[pl.ds(s * _STRIPE, _STRIPE)],
                    agg_out.at[c, pl.ds(s * _STRIPE, _STRIPE)])


@jax.jit
def _sc_agg(h, src3, dst3, srcf, dstf):
    mesh = plsc.VectorSubcoreMesh(core_axis_name="c", subcore_axis_name="s")
    return pl.kernel(
        _sc_agg_body,
        out_type=jax.ShapeDtypeStruct((_SC_NC, _NP, D), jnp.float32),
        mesh=mesh,
        scratch_types=(
            pltpu.VMEM((_SPC, _K), jnp.int32),
            pltpu.VMEM((_SPC, _K), jnp.int32),
            pltpu.VMEM((_K,), jnp.int32),
            pltpu.VMEM((_K,), jnp.int32),
            pltpu.VMEM((_K, D), jnp.float32),
            pltpu.VMEM((_K, D), jnp.float32),
            pltpu.VMEM_SHARED((_NP, D), jnp.float32),
            pltpu.SemaphoreType.DMA,
            pltpu.SemaphoreType.DMA,
        ),
    )(h, src3, dst3, srcf, dstf)


_EB = 12800
_DR = _NP // _K


def _deg_body(dst_ref, o_ref, acc_ref):
    i = pl.program_id(0)

    @pl.when(i == 0)
    def _init():
        acc_ref[...] = jnp.zeros((_DR, _K), jnp.float32)

    d = dst_ref[...]
    q = lax.shift_right_logical(d, 7)
    r = lax.bitwise_and(d, 127)
    oq = (q == lax.broadcasted_iota(jnp.int32, (1, _DR), 1)
          ).astype(jnp.bfloat16)
    orr = (r == lax.broadcasted_iota(jnp.int32, (1, _K), 1)
           ).astype(jnp.bfloat16)
    acc_ref[...] += lax.dot_general(oq, orr, (((0,), (0,)), ((), ())),
                                    preferred_element_type=jnp.float32)

    @pl.when(i == pl.num_programs(0) - 1)
    def _fin():
        o_ref[...] = acc_ref[...]


def _deg(dstc):
    return pl.pallas_call(
        _deg_body,
        grid=(E // _EB,),
        in_specs=[pl.BlockSpec((_EB, 1), lambda i: (i, 0))],
        out_specs=pl.BlockSpec((_DR, _K), lambda i: (0, 0)),
        out_shape=jax.ShapeDtypeStruct((_DR, _K), jnp.float32),
        scratch_shapes=[pltpu.VMEM((_DR, _K), jnp.float32)],
    )(dstc)


def _bn_body(x_ref, g_ref, b_ref, o_ref):
    x = x_ref[...]
    mu = jnp.mean(x, axis=0, keepdims=True)
    var = jnp.mean((x - mu) ** 2, axis=0, keepdims=True)
    o_ref[...] = (x - mu) / jnp.sqrt(var + 1e-5) * g_ref[...] + b_ref[...]


def _bn(x, gamma, beta):
    return pl.pallas_call(
        _bn_body,
        out_shape=jax.ShapeDtypeStruct((N, D), jnp.float32),
    )(x, gamma.reshape(1, D), beta.reshape(1, D))


def _combine_body(p_ref, deg_ref, h_ref, wl_ref, bl_ref, wr_ref, o_ref):
    agg = jnp.sum(p_ref[...], axis=0)
    zl = lax.dot_general(agg, wl_ref[...], (((1,), (1,)), ((), ())),
                         preferred_element_type=jnp.float32)
    zr = lax.dot_general(h_ref[...], wr_ref[...], (((1,), (1,)), ((), ())),
                         preferred_element_type=jnp.float32)
    deg = jnp.maximum(deg_ref[...], 1.0)
    z = zl / deg + bl_ref[...] + zr
    o_ref[...] = jnp.where(z > 0, z, jnp.exp(jnp.minimum(z, 0.0)) - 1.0)


def _combine(partials, deg, h, Wl, bl, Wr):
    P = partials.shape[0]
    grid = N // _ROWS
    return pl.pallas_call(
        _combine_body,
        grid=(grid,),
        in_specs=[
            pl.BlockSpec((P, _ROWS, D), lambda i: (0, i, 0)),
            pl.BlockSpec((_ROWS, 1), lambda i: (i, 0)),
            pl.BlockSpec((_ROWS, D), lambda i: (i, 0)),
            pl.BlockSpec((H, D), lambda i: (0, 0)),
            pl.BlockSpec((1, H), lambda i: (0, 0)),
            pl.BlockSpec((H, D), lambda i: (0, 0)),
        ],
        out_specs=pl.BlockSpec((_ROWS, H), lambda i: (i, 0)),
        out_shape=jax.ShapeDtypeStruct((N, H), jnp.float32),
    )(partials, deg, h, Wl, bl.reshape(1, H), Wr)


def _final_body(p_ref, deg_ref, h_ref, wl_ref, bl_ref, wr_ref,
                batch_ref, wc1_ref, bc1_ref, wc2_ref, bc2_ref,
                out_ref, rv_ref, acc_ref):
    i = pl.program_id(0)
    nsteps = pl.num_programs(0)

    @pl.when(i == 0)
    def _init():
        acc_ref[...] = jnp.full((G, H), -jnp.inf, jnp.float32)

    agg = jnp.sum(p_ref[...], axis=0)
    zl = lax.dot_general(agg, wl_ref[...], (((1,), (1,)), ((), ())),
                         preferred_element_type=jnp.float32)
    zr = lax.dot_general(h_ref[...], wr_ref[...], (((1,), (1,)), ((), ())),
                         preferred_element_type=jnp.float32)
    deg = jnp.maximum(deg_ref[...], 1.0)
    z = zl / deg + bl_ref[...] + zr
    h2 = jnp.where(z > 0, z, jnp.exp(jnp.minimum(z, 0.0)) - 1.0)

    batch = batch_ref[...]
    gmin = jnp.min(batch)
    gmax = jnp.max(batch)

    def body(g, _):
        m = batch == g
        val = jnp.max(jnp.where(m, h2, -jnp.inf), axis=0, keepdims=True)
        cur = acc_ref[pl.ds(g, 1), :]
        acc_ref[pl.ds(g, 1), :] = jnp.maximum(cur, val)
        return 0

    lax.fori_loop(gmin, gmax + 1, body, 0)

    @pl.when(i == nsteps - 1)
    def _fin():
        acc = acc_ref[...]
        read_out = jnp.where(jnp.isfinite(acc), acc, 0.0)
        rv = lax.dot_general(read_out, wc1_ref[...], (((1,), (1,)), ((), ())),
                             preferred_element_type=jnp.float32) + bc1_ref[...]
        out = lax.dot_general(rv, wc2_ref[...], (((1,), (1,)), ((), ())),
                              preferred_element_type=jnp.float32) + bc2_ref[...]
        rv_ref[...] = rv
        out_ref[...] = out


def _final(partials, deg, h, Wl, bl, Wr, batch, Wc1, bc1, Wc2, bc2):
    P = partials.shape[0]
    grid = N // _ROWS
    return pl.pallas_call(
        _final_body,
        grid=(grid,),
        in_specs=[
            pl.BlockSpec((P, _ROWS, D), lambda i: (0, i, 0)),
            pl.BlockSpec((_ROWS, 1), lambda i: (i, 0)),
            pl.BlockSpec((_ROWS, D), lambda i: (i, 0)),
            pl.BlockSpec((H, H), lambda i: (0, 0)),
            pl.BlockSpec((1, H), lambda i: (0, 0)),
            pl.BlockSpec((H, H), lambda i: (0, 0)),
            pl.BlockSpec((_ROWS, 1), lambda i: (i, 0)),
            pl.BlockSpec((S, H), lambda i: (0, 0)),
            pl.BlockSpec((1, S), lambda i: (0, 0)),
            pl.BlockSpec((C, S), lambda i: (0, 0)),
            pl.BlockSpec((1, C), lambda i: (0, 0)),
        ],
        out_specs=[
            pl.BlockSpec((G, C), lambda i: (0, 0)),
            pl.BlockSpec((G, S), lambda i: (0, 0)),
        ],
        out_shape=[
            jax.ShapeDtypeStruct((G, C), jnp.float32),
            jax.ShapeDtypeStruct((G, S), jnp.float32),
        ],
        scratch_shapes=[pltpu.VMEM((G, H), jnp.float32)],
    )(partials, deg, h, Wl, bl.reshape(1, H), Wr, batch,
      Wc1, bc1.reshape(1, S), Wc2, bc2.reshape(1, C))


def kernel(x, edge_index, batch, bn_gamma, bn_beta,
           W1l, b1l, W1r, W2l, b2l, W2r, Wc1, bc1, Wc2, bc2):
    srcf = edge_index[0]
    dstf = edge_index[1]
    main = _VW * _SPC * _K
    src3 = srcf[:main].reshape(_VW, _SPC, _K)
    dst3 = dstf[:main].reshape(_VW, _SPC, _K)

    degf = _deg(dstf[:, None]).reshape(_NP)[:N, None]

    h0 = _bn(x, bn_gamma, bn_beta)
    agg1 = _sc_agg(h0, src3, dst3, srcf, dstf)
    h1 = _combine(agg1, degf, h0, W1l, b1l, W1r)
    agg2 = _sc_agg(h1, src3, dst3, srcf, dstf)
    out, rv = _final(agg2, degf, h1, W2l, b2l, W2r, batch[:, None],
                     Wc1, bc1, Wc2, bc2)
    return (out, rv)
